# Initial kernel scaffold; baseline (speedup 1.0000x reference)
import jax
import jax.numpy as jnp
from jax.experimental import pallas as pl


def kernel(pred, targets):
    raise NotImplementedError("write your pallas kernel here")


# speedup vs baseline: 10.1573x; 10.1573x over previous
import functools

import jax
import jax.numpy as jnp
from jax import lax
from jax.experimental import pallas as pl
from jax.experimental.pallas import tpu as pltpu

_STRIDE = 8
_GAMMA = 2.0
_ALPHA = 0.25
_CLS_WEIGHT = 0.8
_REG_WEIGHT = 0.2

_BS = 16
_H = 128
_W = 128
_NT = 8
_NCELL = _BS * _NT
_NFLAT = _BS * _H * _W
_HUGE_I = 2 ** 30
_NEG_INF = float("-inf")


def _loss_kernel(gxs, gys, offxs, offys, cells, pcls, pox, poy,
                 out_ref, mscore, bcemap, rmaxs):
    yi = lax.broadcasted_iota(jnp.int32, (_H, _W), 0)
    xi = lax.broadcasted_iota(jnp.int32, (_H, _W), 1)
    yif = yi.astype(jnp.float32)
    xif = xi.astype(jnp.float32)
    lane128 = lax.broadcasted_iota(jnp.int32, (1, _W), 1)

    total_focal = jnp.float32(0.0)
    pos_focal = jnp.float32(0.0)
    npos_f = jnp.float32(0.0)
    reg_sum = jnp.float32(0.0)

    for b in range(_BS):
        slab = pcls[b * _H:(b + 1) * _H, :]
        cxf = gxs[b, _NT - 1].astype(jnp.float32)
        cyf = gys[b, _NT - 1].astype(jnp.float32)
        d2 = (xif - cxf) ** 2 + (yif - cyf) ** 2
        t = jnp.exp(-2.0 * d2)
        absp = jnp.abs(slab)
        bce = jnp.maximum(slab, 0.0) - slab * t + jnp.log1p(jnp.exp(-absp))
        p_t = jnp.exp(-bce)
        alpha = t * _ALPHA + (1.0 - t) * (1.0 - _ALPHA)
        one_m = 1.0 - p_t
        focal = alpha * one_m * one_m * bce
        pos = jnp.zeros((_H, _W), jnp.bool_)
        toffx = jnp.zeros((_H, _W), jnp.float32)
        toffy = jnp.zeros((_H, _W), jnp.float32)
        for j in range(_NT):
            hit = (xi == gxs[b, j]) & (yi == gys[b, j])
            pos = pos | hit
            toffx = jnp.where(hit, offxs[b, j], toffx)
            toffy = jnp.where(hit, offys[b, j], toffy)
        posf = pos.astype(jnp.float32)
        score = 1.0 / (1.0 + jnp.exp(-slab))
        msl = jnp.where(pos, _NEG_INF, score)
        mscore[b * _H:(b + 1) * _H, :] = msl
        bcemap[b * _H:(b + 1) * _H, :] = bce
        rmaxs[:, b:b + 1] = jnp.max(msl, axis=1, keepdims=True)
        total_focal += jnp.sum(focal)
        pos_focal += jnp.sum(focal * posf)
        npos_f += jnp.sum(posf)
        pxs = pox[b * _H:(b + 1) * _H, :]
        pys = poy[b * _H:(b + 1) * _H, :]
        reg_sum += jnp.sum(posf * (jnp.abs(pxs - toffx) + jnp.abs(pys - toffy)))

    npos_i = npos_f.astype(jnp.int32)

    cv = cells[:]
    jmod = lax.broadcasted_iota(jnp.int32, (1, _NCELL), 1) % _NT
    dup = jnp.zeros((1, _NCELL), jnp.bool_)
    for d in range(1, _NT):
        shifted = jnp.concatenate(
            [cv[:, d:], jnp.full((1, d), _HUGE_I, jnp.int32)], axis=1)
        dup = dup | ((shifted == cv) & (jmod < _NT - d))
    uniqcells = jnp.where(dup, _HUGE_I, cv)

    rkey = (lax.broadcasted_iota(jnp.int32, (_H, _BS), 1) * _H
            + lax.broadcasted_iota(jnp.int32, (_H, _BS), 0))

    def body(i, carry):
        rmax, hard_sum = carry
        m = jnp.max(rmax)
        r = jnp.min(jnp.where(rmax == m, rkey, _HUGE_I))
        row = mscore[pl.ds(r, 1), :]
        m2 = jnp.max(row)
        x = jnp.min(jnp.where(row == m2, lane128, _HUGE_I))
        p = r * _W + x
        posle = jnp.sum((uniqcells <= p).astype(jnp.int32))
        c = p - posle
        brow = bcemap[pl.ds(c // _W, 1), :]
        val = jnp.sum(jnp.where(lane128 == (c % _W), brow, 0.0))
        hard_sum += jnp.where(i < npos_i, val, 0.0)
        nrow = jnp.where(lane128 == x, _NEG_INF, row)
        mscore[pl.ds(r, 1), :] = nrow
        nrm = jnp.max(nrow)
        rmax = jnp.where((rkey == r), nrm, rmax)
        return rmax, hard_sum

    rmax0 = rmaxs[:, :]
    _, hard_sum = lax.fori_loop(0, _NCELL, body, (rmax0, jnp.float32(0.0)))

    nneg_f = jnp.float32(_NFLAT) - npos_f
    den_pos = jnp.maximum(npos_f, 1.0)
    pos_loss = pos_focal / den_pos
    neg_loss = (total_focal - pos_focal) / jnp.maximum(nneg_f, 1.0)
    hard_neg_loss = hard_sum / den_pos
    cls_loss = pos_loss + neg_loss + hard_neg_loss
    reg = reg_sum / (den_pos * 2.0)
    reg_loss = jnp.where(npos_i == 0, 0.0, reg / den_pos)
    out_ref[0, 0] = _CLS_WEIGHT * cls_loss + _REG_WEIGHT * reg_loss


@functools.partial(jax.jit, static_argnames=("interpret",))
def _run(pred, targets, interpret=False):
    t0 = targets[:, :, 0]
    t1 = targets[:, :, 1]
    gx = jnp.minimum(t0 // _STRIDE, _W - 1)
    gy = jnp.minimum(t1 // _STRIDE, _H - 1)
    offx = (t0.astype(jnp.float32)
            - (gx * _STRIDE).astype(jnp.float32)) / _STRIDE
    offy = (t1.astype(jnp.float32)
            - (gy * _STRIDE).astype(jnp.float32)) / _STRIDE
    cells = (jnp.arange(_BS, dtype=jnp.int32)[:, None] * (_H * _W)
             + gy * _W + gx).reshape(1, _NCELL)
    pr = pred.reshape(_BS * _H * _W, 3)
    pox = pr[:, 0].reshape(_BS * _H, _W)
    poy = pr[:, 1].reshape(_BS * _H, _W)
    pcls = pr[:, 2].reshape(_BS * _H, _W)

    smem = pl.BlockSpec(memory_space=pltpu.SMEM)
    vmem = pl.BlockSpec(memory_space=pltpu.VMEM)
    out = pl.pallas_call(
        _loss_kernel,
        out_shape=jax.ShapeDtypeStruct((1, 1), jnp.float32),
        in_specs=[smem, smem, smem, smem, vmem, vmem, vmem, vmem],
        out_specs=pl.BlockSpec(memory_space=pltpu.SMEM),
        scratch_shapes=[
            pltpu.VMEM((_BS * _H, _W), jnp.float32),
            pltpu.VMEM((_BS * _H, _W), jnp.float32),
            pltpu.VMEM((_H, _BS), jnp.float32),
        ],
        interpret=interpret,
    )(gx, gy, offx, offy, cells, pcls, pox, poy)
    return out[0, 0]


def kernel(pred, targets):
    return _run(pred, targets)


# speedup vs baseline: 17.5510x; 1.7279x over previous
import functools

import jax
import jax.numpy as jnp
from jax import lax
from jax.experimental import pallas as pl
from jax.experimental.pallas import tpu as pltpu
from jax.experimental.pallas import tpu_sc as plsc

_STRIDE = 8
_ALPHA = 0.25
_CLS_WEIGHT = 0.8
_REG_WEIGHT = 0.2

_BS = 16
_H = 128
_W = 128
_NT = 8
_NCELL = _BS * _NT
_NFLAT = _BS * _H * _W
_HUGE_I = 2 ** 30
_NEG_INF = float("-inf")

_NC = 2
_NS = 16
_NW = _NC * _NS
_CHUNK = _NFLAT // _NW
_VROWS = _CHUNK // 16


def _dense_kernel(gxs, gys, offxs, offys, cells, pcls, pox, poy,
                  skeys, bcemap, ucells_out, params, sums, c0s):
    yi = lax.broadcasted_iota(jnp.int32, (_H, _W), 0)
    xi = lax.broadcasted_iota(jnp.int32, (_H, _W), 1)
    yif = yi.astype(jnp.float32)
    xif = xi.astype(jnp.float32)

    total_focal = jnp.float32(0.0)
    pos_focal = jnp.float32(0.0)
    npos_f = jnp.float32(0.0)
    reg_sum = jnp.float32(0.0)

    for b in range(_BS):
        slab = pcls[b * _H:(b + 1) * _H, :]
        cxf = gxs[b, _NT - 1].astype(jnp.float32)
        cyf = gys[b, _NT - 1].astype(jnp.float32)
        d2 = (xif - cxf) ** 2 + (yif - cyf) ** 2
        t = jnp.exp(-2.0 * d2)
        absp = jnp.abs(slab)
        bce = jnp.maximum(slab, 0.0) - slab * t + jnp.log1p(jnp.exp(-absp))
        p_t = jnp.exp(-bce)
        alpha = t * _ALPHA + (1.0 - t) * (1.0 - _ALPHA)
        one_m = 1.0 - p_t
        focal = alpha * one_m * one_m * bce
        pos = jnp.zeros((_H, _W), jnp.bool_)
        toffx = jnp.zeros((_H, _W), jnp.float32)
        toffy = jnp.zeros((_H, _W), jnp.float32)
        for j in range(_NT):
            hit = (xi == gxs[b, j]) & (yi == gys[b, j])
            pos = pos | hit
            toffx = jnp.where(hit, offxs[b, j], toffx)
            toffy = jnp.where(hit, offys[b, j], toffy)
        posf = pos.astype(jnp.float32)
        score = 1.0 / (1.0 + jnp.exp(-slab))
        msl = jnp.where(pos, _NEG_INF, score)
        ibits = lax.bitcast_convert_type(msl, jnp.int32)
        skeys[b * _H:(b + 1) * _H, :] = jnp.maximum(
            jnp.where(ibits >= 0, ibits, ibits ^ 0x7FFFFFFF), -1)
        bcemap[b * _H:(b + 1) * _H, :] = bce
        total_focal += jnp.sum(focal)
        pos_focal += jnp.sum(focal * posf)
        npos_f += jnp.sum(posf)
        pxs = pox[b * _H:(b + 1) * _H, :]
        pys = poy[b * _H:(b + 1) * _H, :]
        reg_sum += jnp.sum(posf * (jnp.abs(pxs - toffx) + jnp.abs(pys - toffy)))

    npos_i = npos_f.astype(jnp.int32)

    cv = cells[:]
    jmod = lax.broadcasted_iota(jnp.int32, (1, _NCELL), 1) % _NT
    dup = jnp.zeros((1, _NCELL), jnp.bool_)
    for d in range(1, _NT):
        shifted = jnp.concatenate(
            [cv[:, d:], jnp.full((1, d), _HUGE_I, jnp.int32)], axis=1)
        dup = dup | ((shifted == cv) & (jmod < _NT - d))
    uniqcells = jnp.where(dup, _HUGE_I, cv)
    ucells_out[:, :] = uniqcells

    skv = skeys[:, :]
    k = npos_i

    def vbody(i, tv):
        cand = tv + lax.shift_left(jnp.int32(1), 30 - i)
        c = jnp.sum((skv >= cand).astype(jnp.int32))
        return jnp.where(c >= k, cand, tv)

    tv = lax.fori_loop(0, 31, vbody, jnp.int32(0))

    count_gt = jnp.sum((skv >= tv + 1).astype(jnp.int32))
    need = k - count_gt

    pgrid = (lax.broadcasted_iota(jnp.int32, (_BS * _H, _W), 0) * _W
             + lax.broadcasted_iota(jnp.int32, (_BS * _H, _W), 1))
    eq = skv == tv

    def ibody(i, iv):
        cand = iv + lax.shift_left(jnp.int32(1), 17 - i)
        c = jnp.sum((eq & (pgrid <= cand)).astype(jnp.int32))
        return jnp.where(c < need, cand, iv)

    ismall = lax.fori_loop(0, 18, ibody, jnp.int32(-1))
    ti = ismall + 1

    for w in range(_NW):
        c0s[w] = jnp.sum((uniqcells < w * _CHUNK).astype(jnp.int32))

    params[0] = tv
    params[1] = ti
    for q in range(2, 16):
        params[q] = 0
    sums[0] = total_focal
    sums[1] = pos_focal
    sums[2] = npos_f
    sums[3] = reg_sum
    for q in range(4, 8):
        sums[q] = 0.0


@functools.cache
def _make_sc_hard():
  mesh = plsc.VectorSubcoreMesh(
      core_axis_name="c", subcore_axis_name="s",
      num_cores=_NC, num_subcores=_NS)

  halo = 128

  @functools.partial(
      pl.kernel,
      out_type=jax.ShapeDtypeStruct((_NW, 16), jnp.float32),
      mesh=mesh,
      compiler_params=pltpu.CompilerParams(needs_layout_passes=False),
      scratch_types=[
          pltpu.VMEM((_CHUNK,), jnp.int32),
          pltpu.VMEM((_CHUNK + halo,), jnp.float32),
          pltpu.VMEM((16 * 128,), jnp.int32),
          pltpu.VMEM((_NT * 16,), jnp.int32),
          pltpu.VMEM((16,), jnp.int32),
          pltpu.VMEM((32,), jnp.int32),
          pltpu.VMEM((16,), jnp.float32),
      ],
  )
  def _sc_hard(skeys_hbm, bce_hbm, tcells_hbm, c0_hbm, params_hbm, out_hbm,
               chunk_v, bce_v, sel_v, tcells_v, c0_v, params_v, part_v):
    cid = lax.axis_index("c")
    sid = lax.axis_index("s")
    wid = sid * _NC + cid
    base = pl.multiple_of(wid * _CHUNK, 8)

    pltpu.sync_copy(skeys_hbm.at[pl.ds(base, _CHUNK)], chunk_v)
    lstart = pl.multiple_of(jnp.maximum(base - halo, 0), 8)
    pltpu.sync_copy(bce_hbm.at[pl.ds(lstart, halo)], bce_v.at[pl.ds(0, halo)])
    pltpu.sync_copy(bce_hbm.at[pl.ds(base, _CHUNK)],
                    bce_v.at[pl.ds(halo, _CHUNK)])
    tcb = pl.multiple_of(wid * (_NT * 16), 8)
    pltpu.sync_copy(tcells_hbm.at[pl.ds(tcb, _NT * 16)], tcells_v)
    c0b = pl.multiple_of(wid * 16, 8)
    pltpu.sync_copy(c0_hbm.at[pl.ds(c0b, 16)], c0_v)
    pltpu.sync_copy(params_hbm, params_v)

    def sbody(i, lanecnt):
      it = lax.iota(jnp.int32, 16)
      tvl = params_v[pl.ds(0, 16)]
      til = params_v[pl.ds(16, 16)]
      v = chunk_v[pl.ds(pl.multiple_of(i * 16, 16), 16)]
      pv = base + i * 16 + it
      gt = jnp.where(v > tvl, 1, 0)
      eq = jnp.where(v == tvl, 1, 0)
      le = jnp.where(pv <= til, 1, 0)
      seli = gt + eq * le
      plsc.store_scatter(sel_v, [it * 128 + lanecnt], pv, mask=(seli == 1))
      return lanecnt + seli

    lanecnt = lax.fori_loop(0, _VROWS, sbody, jnp.zeros((16,), jnp.int32))

    off = base - halo

    def gbody(s, carry):
      psum, lcv = carry
      it = lax.iota(jnp.int32, 16)
      pv = plsc.load_gather(sel_v, [it * 128 + s])
      valid = lcv > s
      posle = c0_v[...]
      for r in range(_NT):
        cr = tcells_v[pl.ds(r * 16, 16)]
        posle = posle + jnp.where(pv >= cr, 1, 0)
      lidx = jnp.where(valid, pv - posle - off, 0)
      vals = plsc.load_gather(bce_v, [lidx])
      return psum + jnp.where(valid, vals, 0.0), lcv

    psum, _ = lax.fori_loop(
        0, 128, gbody, (jnp.zeros((16,), jnp.float32), lanecnt))

    part_v[...] = psum
    pltpu.sync_copy(part_v, out_hbm.at[wid])

  return _sc_hard


@jax.jit
def _run(pred, targets):
    t0 = targets[:, :, 0]
    t1 = targets[:, :, 1]
    gx = jnp.minimum(t0 // _STRIDE, _W - 1)
    gy = jnp.minimum(t1 // _STRIDE, _H - 1)
    offx = (t0.astype(jnp.float32)
            - (gx * _STRIDE).astype(jnp.float32)) / _STRIDE
    offy = (t1.astype(jnp.float32)
            - (gy * _STRIDE).astype(jnp.float32)) / _STRIDE
    cells = (jnp.arange(_BS, dtype=jnp.int32)[:, None] * (_H * _W)
             + gy * _W + gx).reshape(1, _NCELL)
    pr = pred.reshape(_BS * _H * _W, 3)
    pox = pr[:, 0].reshape(_BS * _H, _W)
    poy = pr[:, 1].reshape(_BS * _H, _W)
    pcls = pr[:, 2].reshape(_BS * _H, _W)

    smem = pl.BlockSpec(memory_space=pltpu.SMEM)
    vmem = pl.BlockSpec(memory_space=pltpu.VMEM)
    skeys, bcemap, ucells, params, sums, c0s = pl.pallas_call(
        _dense_kernel,
        out_shape=[
            jax.ShapeDtypeStruct((_BS * _H, _W), jnp.int32),
            jax.ShapeDtypeStruct((_BS * _H, _W), jnp.float32),
            jax.ShapeDtypeStruct((1, _NCELL), jnp.int32),
            jax.ShapeDtypeStruct((16,), jnp.int32),
            jax.ShapeDtypeStruct((8,), jnp.float32),
            jax.ShapeDtypeStruct((_NW,), jnp.int32),
        ],
        in_specs=[smem, smem, smem, smem, vmem, vmem, vmem, vmem],
        out_specs=[vmem, vmem, vmem, smem, smem, smem],
    )(gx, gy, offx, offy, cells, pcls, pox, poy)

    u = ucells.reshape(_BS, _NT)
    rel = u - jnp.arange(_BS, dtype=jnp.int32)[:, None] * (_H * _W)
    tc_lo = jnp.where((rel >= 0) & (rel < _CHUNK), u, _HUGE_I)
    tc_hi = jnp.where((rel >= _CHUNK) & (rel < 2 * _CHUNK), u, _HUGE_I)
    tcells = jnp.stack([tc_lo, tc_hi], axis=1)
    tcells_splat = jnp.repeat(tcells.reshape(_NW * _NT), 16)
    c0_splat = jnp.repeat(c0s, 16)
    params_splat = jnp.repeat(params[:2], 16)
    hard_parts = _make_sc_hard()(
        skeys.reshape(_NFLAT), bcemap.reshape(_NFLAT),
        tcells_splat, c0_splat, params_splat)
    hard_sum = jnp.sum(hard_parts)

    total_focal = sums[0]
    pos_focal = sums[1]
    npos_f = sums[2]
    reg_sum = sums[3]
    nneg_f = jnp.float32(_NFLAT) - npos_f
    den_pos = jnp.maximum(npos_f, 1.0)
    pos_loss = pos_focal / den_pos
    neg_loss = (total_focal - pos_focal) / jnp.maximum(nneg_f, 1.0)
    hard_neg_loss = hard_sum / den_pos
    cls_loss = pos_loss + neg_loss + hard_neg_loss
    reg = reg_sum / (den_pos * 2.0)
    reg_loss = jnp.where(npos_f == 0.0, 0.0, reg / den_pos)
    return _CLS_WEIGHT * cls_loss + _REG_WEIGHT * reg_loss


def kernel(pred, targets):
    return _run(pred, targets)


# speedup vs baseline: 19.1701x; 1.0923x over previous
import functools

import jax
import jax.numpy as jnp
from jax import lax
from jax.experimental import pallas as pl
from jax.experimental.pallas import tpu as pltpu
from jax.experimental.pallas import tpu_sc as plsc

_STRIDE = 8
_ALPHA = 0.25
_CLS_WEIGHT = 0.8
_REG_WEIGHT = 0.2

_BS = 16
_H = 128
_W = 128
_NT = 8
_NCELL = _BS * _NT
_NFLAT = _BS * _H * _W
_HUGE_I = 2 ** 30
_NEG_INF = float("-inf")

_NC = 2
_NS = 16
_NW = _NC * _NS
_CHUNK = _NFLAT // _NW
_VROWS = _CHUNK // 16


def _dense_kernel(gxs, gys, offxs, offys, cells, pcls, pox, poy,
                  skeys, bcemap, ucells_out, params, sums, c0s):
    yi = lax.broadcasted_iota(jnp.int32, (_H, _W), 0)
    xi = lax.broadcasted_iota(jnp.int32, (_H, _W), 1)
    yif = yi.astype(jnp.float32)
    xif = xi.astype(jnp.float32)

    total_focal = jnp.float32(0.0)
    pos_focal = jnp.float32(0.0)
    npos_f = jnp.float32(0.0)
    reg_sum = jnp.float32(0.0)

    for b in range(_BS):
        slab = pcls[b * _H:(b + 1) * _H, :]
        cxf = gxs[b, _NT - 1].astype(jnp.float32)
        cyf = gys[b, _NT - 1].astype(jnp.float32)
        d2 = (xif - cxf) ** 2 + (yif - cyf) ** 2
        t = jnp.exp(-2.0 * d2)
        absp = jnp.abs(slab)
        bce = jnp.maximum(slab, 0.0) - slab * t + jnp.log1p(jnp.exp(-absp))
        p_t = jnp.exp(-bce)
        alpha = t * _ALPHA + (1.0 - t) * (1.0 - _ALPHA)
        one_m = 1.0 - p_t
        focal = alpha * one_m * one_m * bce
        pos = jnp.zeros((_H, _W), jnp.bool_)
        toffx = jnp.zeros((_H, _W), jnp.float32)
        toffy = jnp.zeros((_H, _W), jnp.float32)
        for j in range(_NT):
            hit = (xi == gxs[b, j]) & (yi == gys[b, j])
            pos = pos | hit
            toffx = jnp.where(hit, offxs[b, j], toffx)
            toffy = jnp.where(hit, offys[b, j], toffy)
        posf = pos.astype(jnp.float32)
        score = 1.0 / (1.0 + jnp.exp(-slab))
        msl = jnp.where(pos, _NEG_INF, score)
        ibits = lax.bitcast_convert_type(msl, jnp.int32)
        skeys[b * _H:(b + 1) * _H, :] = jnp.maximum(
            jnp.where(ibits >= 0, ibits, ibits ^ 0x7FFFFFFF), -1)
        bcemap[b * _H:(b + 1) * _H, :] = bce
        total_focal += jnp.sum(focal)
        pos_focal += jnp.sum(focal * posf)
        npos_f += jnp.sum(posf)
        pxs = pox[b * _H:(b + 1) * _H, :]
        pys = poy[b * _H:(b + 1) * _H, :]
        reg_sum += jnp.sum(posf * (jnp.abs(pxs - toffx) + jnp.abs(pys - toffy)))

    npos_i = npos_f.astype(jnp.int32)

    cv = cells[:]
    jmod = lax.broadcasted_iota(jnp.int32, (1, _NCELL), 1) % _NT
    dup = jnp.zeros((1, _NCELL), jnp.bool_)
    for d in range(1, _NT):
        shifted = jnp.concatenate(
            [cv[:, d:], jnp.full((1, d), _HUGE_I, jnp.int32)], axis=1)
        dup = dup | ((shifted == cv) & (jmod < _NT - d))
    uniqcells = jnp.where(dup, _HUGE_I, cv)
    ucells_out[:, :] = uniqcells

    skv = skeys[:, :]
    k = npos_i

    def vbody(i, tv):
        cand = tv + lax.shift_left(jnp.int32(1), 30 - i)
        c = jnp.sum((skv >= cand).astype(jnp.int32))
        return jnp.where(c >= k, cand, tv)

    tv = lax.fori_loop(0, 31, vbody, jnp.int32(0))

    count_gt = jnp.sum((skv >= tv + 1).astype(jnp.int32))
    need = k - count_gt

    pgrid = (lax.broadcasted_iota(jnp.int32, (_BS * _H, _W), 0) * _W
             + lax.broadcasted_iota(jnp.int32, (_BS * _H, _W), 1))
    eq = skv == tv

    def ibody(i, iv):
        cand = iv + lax.shift_left(jnp.int32(1), 17 - i)
        c = jnp.sum((eq & (pgrid <= cand)).astype(jnp.int32))
        return jnp.where(c < need, cand, iv)

    count_ge = jnp.sum((skv >= tv).astype(jnp.int32))
    n_idx_passes = jnp.where(count_ge == k, 0, 18)
    ismall = lax.fori_loop(0, n_idx_passes, ibody, jnp.int32(-1))
    ti = jnp.where(count_ge == k, jnp.int32(_NFLAT), ismall + 1)

    for w in range(_NW):
        c0s[w] = jnp.sum((uniqcells < w * _CHUNK).astype(jnp.int32))

    params[0] = tv
    params[1] = ti
    for q in range(2, 16):
        params[q] = 0
    sums[0] = total_focal
    sums[1] = pos_focal
    sums[2] = npos_f
    sums[3] = reg_sum
    for q in range(4, 8):
        sums[q] = 0.0


@functools.cache
def _make_sc_hard():
  mesh = plsc.VectorSubcoreMesh(
      core_axis_name="c", subcore_axis_name="s",
      num_cores=_NC, num_subcores=_NS)

  halo = 128

  @functools.partial(
      pl.kernel,
      out_type=jax.ShapeDtypeStruct((_NW, 16), jnp.float32),
      mesh=mesh,
      compiler_params=pltpu.CompilerParams(needs_layout_passes=False),
      scratch_types=[
          pltpu.VMEM((_CHUNK,), jnp.int32),
          pltpu.VMEM((_CHUNK + halo,), jnp.float32),
          pltpu.VMEM((16 * 128,), jnp.int32),
          pltpu.VMEM((_NT * 16,), jnp.int32),
          pltpu.VMEM((16,), jnp.int32),
          pltpu.VMEM((32,), jnp.int32),
          pltpu.VMEM((16,), jnp.float32),
      ],
  )
  def _sc_hard(skeys_hbm, bce_hbm, tcells_hbm, c0_hbm, params_hbm, out_hbm,
               chunk_v, bce_v, sel_v, tcells_v, c0_v, params_v, part_v):
    cid = lax.axis_index("c")
    sid = lax.axis_index("s")
    wid = sid * _NC + cid
    base = pl.multiple_of(wid * _CHUNK, 8)

    pltpu.sync_copy(skeys_hbm.at[pl.ds(base, _CHUNK)], chunk_v)
    lstart = pl.multiple_of(jnp.maximum(base - halo, 0), 8)
    pltpu.sync_copy(bce_hbm.at[pl.ds(lstart, halo)], bce_v.at[pl.ds(0, halo)])
    pltpu.sync_copy(bce_hbm.at[pl.ds(base, _CHUNK)],
                    bce_v.at[pl.ds(halo, _CHUNK)])
    tcb = pl.multiple_of(wid * (_NT * 16), 8)
    pltpu.sync_copy(tcells_hbm.at[pl.ds(tcb, _NT * 16)], tcells_v)
    c0b = pl.multiple_of(wid * 16, 8)
    pltpu.sync_copy(c0_hbm.at[pl.ds(c0b, 16)], c0_v)
    pltpu.sync_copy(params_hbm, params_v)

    def sbody(i, lanecnt):
      it = lax.iota(jnp.int32, 16)
      tvl = params_v[pl.ds(0, 16)]
      til = params_v[pl.ds(16, 16)]
      v = chunk_v[pl.ds(pl.multiple_of(i * 16, 16), 16)]
      pv = base + i * 16 + it
      gt = jnp.where(v > tvl, 1, 0)
      eq = jnp.where(v == tvl, 1, 0)
      le = jnp.where(pv <= til, 1, 0)
      seli = gt + eq * le
      plsc.store_scatter(sel_v, [it * 128 + lanecnt], pv, mask=(seli == 1))
      return lanecnt + seli

    lanecnt = lax.fori_loop(0, _VROWS, sbody, jnp.zeros((16,), jnp.int32))

    off = base - halo

    def gbody(s, carry):
      psum, lcv = carry
      it = lax.iota(jnp.int32, 16)
      pv = plsc.load_gather(sel_v, [it * 128 + s])
      valid = lcv > s
      posle = c0_v[...]
      for r in range(_NT):
        cr = tcells_v[pl.ds(r * 16, 16)]
        posle = posle + jnp.where(pv >= cr, 1, 0)
      lidx = jnp.where(valid, pv - posle - off, 0)
      vals = plsc.load_gather(bce_v, [lidx])
      return psum + jnp.where(valid, vals, 0.0), lcv

    psum, _ = lax.fori_loop(
        0, 128, gbody, (jnp.zeros((16,), jnp.float32), lanecnt))

    part_v[...] = psum
    pltpu.sync_copy(part_v, out_hbm.at[wid])

  return _sc_hard


@jax.jit
def _run(pred, targets):
    t0 = targets[:, :, 0]
    t1 = targets[:, :, 1]
    gx = jnp.minimum(t0 // _STRIDE, _W - 1)
    gy = jnp.minimum(t1 // _STRIDE, _H - 1)
    offx = (t0.astype(jnp.float32)
            - (gx * _STRIDE).astype(jnp.float32)) / _STRIDE
    offy = (t1.astype(jnp.float32)
            - (gy * _STRIDE).astype(jnp.float32)) / _STRIDE
    cells = (jnp.arange(_BS, dtype=jnp.int32)[:, None] * (_H * _W)
             + gy * _W + gx).reshape(1, _NCELL)
    pr = pred.reshape(_BS * _H * _W, 3)
    pox = pr[:, 0].reshape(_BS * _H, _W)
    poy = pr[:, 1].reshape(_BS * _H, _W)
    pcls = pr[:, 2].reshape(_BS * _H, _W)

    smem = pl.BlockSpec(memory_space=pltpu.SMEM)
    vmem = pl.BlockSpec(memory_space=pltpu.VMEM)
    skeys, bcemap, ucells, params, sums, c0s = pl.pallas_call(
        _dense_kernel,
        out_shape=[
            jax.ShapeDtypeStruct((_BS * _H, _W), jnp.int32),
            jax.ShapeDtypeStruct((_BS * _H, _W), jnp.float32),
            jax.ShapeDtypeStruct((1, _NCELL), jnp.int32),
            jax.ShapeDtypeStruct((16,), jnp.int32),
            jax.ShapeDtypeStruct((8,), jnp.float32),
            jax.ShapeDtypeStruct((_NW,), jnp.int32),
        ],
        in_specs=[smem, smem, smem, smem, vmem, vmem, vmem, vmem],
        out_specs=[vmem, vmem, vmem, smem, smem, smem],
    )(gx, gy, offx, offy, cells, pcls, pox, poy)

    u = ucells.reshape(_BS, _NT)
    rel = u - jnp.arange(_BS, dtype=jnp.int32)[:, None] * (_H * _W)
    tc_lo = jnp.where((rel >= 0) & (rel < _CHUNK), u, _HUGE_I)
    tc_hi = jnp.where((rel >= _CHUNK) & (rel < 2 * _CHUNK), u, _HUGE_I)
    tcells = jnp.stack([tc_lo, tc_hi], axis=1)
    tcells_splat = jnp.repeat(tcells.reshape(_NW * _NT), 16)
    c0_splat = jnp.repeat(c0s, 16)
    params_splat = jnp.repeat(params[:2], 16)
    hard_parts = _make_sc_hard()(
        skeys.reshape(_NFLAT), bcemap.reshape(_NFLAT),
        tcells_splat, c0_splat, params_splat)
    hard_sum = jnp.sum(hard_parts)

    total_focal = sums[0]
    pos_focal = sums[1]
    npos_f = sums[2]
    reg_sum = sums[3]
    nneg_f = jnp.float32(_NFLAT) - npos_f
    den_pos = jnp.maximum(npos_f, 1.0)
    pos_loss = pos_focal / den_pos
    neg_loss = (total_focal - pos_focal) / jnp.maximum(nneg_f, 1.0)
    hard_neg_loss = hard_sum / den_pos
    cls_loss = pos_loss + neg_loss + hard_neg_loss
    reg = reg_sum / (den_pos * 2.0)
    reg_loss = jnp.where(npos_f == 0.0, 0.0, reg / den_pos)
    return _CLS_WEIGHT * cls_loss + _REG_WEIGHT * reg_loss


def kernel(pred, targets):
    return _run(pred, targets)


# speedup vs baseline: 19.5059x; 1.0175x over previous
import functools

import jax
import jax.numpy as jnp
from jax import lax
from jax.experimental import pallas as pl
from jax.experimental.pallas import tpu as pltpu
from jax.experimental.pallas import tpu_sc as plsc

_STRIDE = 8
_ALPHA = 0.25
_CLS_WEIGHT = 0.8
_REG_WEIGHT = 0.2

_BS = 16
_H = 128
_W = 128
_NT = 8
_NCELL = _BS * _NT
_NFLAT = _BS * _H * _W
_HUGE_I = 2 ** 30
_NEG_INF = float("-inf")

_NC = 2
_NS = 16
_NW = _NC * _NS
_CHUNK = _NFLAT // _NW
_VROWS = _CHUNK // 16


def _dense_kernel(gxs, gys, offxs, offys, cells, pcls, pox, poy,
                  skeys, bcemap, ucells_out, params, sums, c0s):
    yi = lax.broadcasted_iota(jnp.int32, (_H, _W), 0)
    xi = lax.broadcasted_iota(jnp.int32, (_H, _W), 1)
    yif = yi.astype(jnp.float32)
    xif = xi.astype(jnp.float32)

    total_focal = jnp.float32(0.0)
    pos_focal = jnp.float32(0.0)
    npos_f = jnp.float32(0.0)
    reg_sum = jnp.float32(0.0)

    for b in range(_BS):
        slab = pcls[b * _H:(b + 1) * _H, :]
        cxf = gxs[b, _NT - 1].astype(jnp.float32)
        cyf = gys[b, _NT - 1].astype(jnp.float32)
        d2 = (xif - cxf) ** 2 + (yif - cyf) ** 2
        t = jnp.exp(-2.0 * d2)
        absp = jnp.abs(slab)
        bce = jnp.maximum(slab, 0.0) - slab * t + jnp.log1p(jnp.exp(-absp))
        p_t = jnp.exp(-bce)
        alpha = t * _ALPHA + (1.0 - t) * (1.0 - _ALPHA)
        one_m = 1.0 - p_t
        focal = alpha * one_m * one_m * bce
        pos = jnp.zeros((_H, _W), jnp.bool_)
        toffx = jnp.zeros((_H, _W), jnp.float32)
        toffy = jnp.zeros((_H, _W), jnp.float32)
        for j in range(_NT):
            hit = (xi == gxs[b, j]) & (yi == gys[b, j])
            pos = pos | hit
            toffx = jnp.where(hit, offxs[b, j], toffx)
            toffy = jnp.where(hit, offys[b, j], toffy)
        posf = pos.astype(jnp.float32)
        score = 1.0 / (1.0 + jnp.exp(-slab))
        msl = jnp.where(pos, _NEG_INF, score)
        ibits = lax.bitcast_convert_type(msl, jnp.int32)
        skeys[b * _H:(b + 1) * _H, :] = jnp.maximum(
            jnp.where(ibits >= 0, ibits, ibits ^ 0x7FFFFFFF), -1)
        bcemap[b * _H:(b + 1) * _H, :] = bce
        total_focal += jnp.sum(focal)
        pos_focal += jnp.sum(focal * posf)
        npos_f += jnp.sum(posf)
        pxs = pox[b * _H:(b + 1) * _H, :]
        pys = poy[b * _H:(b + 1) * _H, :]
        reg_sum += jnp.sum(posf * (jnp.abs(pxs - toffx) + jnp.abs(pys - toffy)))

    npos_i = npos_f.astype(jnp.int32)

    cv = cells[:]
    jmod = lax.broadcasted_iota(jnp.int32, (1, _NCELL), 1) % _NT
    dup = jnp.zeros((1, _NCELL), jnp.bool_)
    for d in range(1, _NT):
        shifted = jnp.concatenate(
            [cv[:, d:], jnp.full((1, d), _HUGE_I, jnp.int32)], axis=1)
        dup = dup | ((shifted == cv) & (jmod < _NT - d))
    uniqcells = jnp.where(dup, _HUGE_I, cv)
    ucells_out[:, :] = uniqcells

    skv = skeys[:, :]
    k = npos_i

    def vbody(i, tv):
        cand = tv + lax.shift_left(jnp.int32(1), 29 - i)
        c = jnp.sum((skv >= cand).astype(jnp.int32))
        return jnp.where(c >= k, cand, tv)

    tv = lax.fori_loop(0, 30, vbody, jnp.int32(0))

    count_gt = jnp.sum((skv >= tv + 1).astype(jnp.int32))
    need = k - count_gt

    pgrid = (lax.broadcasted_iota(jnp.int32, (_BS * _H, _W), 0) * _W
             + lax.broadcasted_iota(jnp.int32, (_BS * _H, _W), 1))
    eq = skv == tv

    def ibody(i, iv):
        cand = iv + lax.shift_left(jnp.int32(1), 17 - i)
        c = jnp.sum((eq & (pgrid <= cand)).astype(jnp.int32))
        return jnp.where(c < need, cand, iv)

    count_ge = jnp.sum((skv >= tv).astype(jnp.int32))
    n_idx_passes = jnp.where(count_ge == k, 0, 18)
    ismall = lax.fori_loop(0, n_idx_passes, ibody, jnp.int32(-1))
    ti = jnp.where(count_ge == k, jnp.int32(_NFLAT), ismall + 1)

    for w in range(_NW):
        c0s[w] = jnp.sum((uniqcells < w * _CHUNK).astype(jnp.int32))

    params[0] = tv
    params[1] = ti
    for q in range(2, 16):
        params[q] = 0
    sums[0] = total_focal
    sums[1] = pos_focal
    sums[2] = npos_f
    sums[3] = reg_sum
    for q in range(4, 8):
        sums[q] = 0.0


@functools.cache
def _make_sc_hard():
  mesh = plsc.VectorSubcoreMesh(
      core_axis_name="c", subcore_axis_name="s",
      num_cores=_NC, num_subcores=_NS)

  halo = 128

  @functools.partial(
      pl.kernel,
      out_type=jax.ShapeDtypeStruct((_NW, 16), jnp.float32),
      mesh=mesh,
      compiler_params=pltpu.CompilerParams(needs_layout_passes=False),
      scratch_types=[
          pltpu.VMEM((_CHUNK,), jnp.int32),
          pltpu.VMEM((_CHUNK + halo,), jnp.float32),
          pltpu.VMEM((16 * 128,), jnp.int32),
          pltpu.VMEM((_NT * 16,), jnp.int32),
          pltpu.VMEM((16,), jnp.int32),
          pltpu.VMEM((32,), jnp.int32),
          pltpu.VMEM((16,), jnp.float32),
      ],
  )
  def _sc_hard(skeys_hbm, bce_hbm, tcells_hbm, c0_hbm, params_hbm, out_hbm,
               chunk_v, bce_v, sel_v, tcells_v, c0_v, params_v, part_v):
    cid = lax.axis_index("c")
    sid = lax.axis_index("s")
    wid = sid * _NC + cid
    base = pl.multiple_of(wid * _CHUNK, 8)

    pltpu.sync_copy(skeys_hbm.at[pl.ds(base, _CHUNK)], chunk_v)
    lstart = pl.multiple_of(jnp.maximum(base - halo, 0), 8)
    pltpu.sync_copy(bce_hbm.at[pl.ds(lstart, halo)], bce_v.at[pl.ds(0, halo)])
    pltpu.sync_copy(bce_hbm.at[pl.ds(base, _CHUNK)],
                    bce_v.at[pl.ds(halo, _CHUNK)])
    tcb = pl.multiple_of(wid * (_NT * 16), 8)
    pltpu.sync_copy(tcells_hbm.at[pl.ds(tcb, _NT * 16)], tcells_v)
    c0b = pl.multiple_of(wid * 16, 8)
    pltpu.sync_copy(c0_hbm.at[pl.ds(c0b, 16)], c0_v)
    pltpu.sync_copy(params_hbm, params_v)

    def sbody(i, lanecnt):
      it = lax.iota(jnp.int32, 16)
      tvl = params_v[pl.ds(0, 16)]
      til = params_v[pl.ds(16, 16)]
      for u in range(2):
        v = chunk_v[pl.ds(pl.multiple_of((2 * i + u) * 16, 16), 16)]
        pv = base + (2 * i + u) * 16 + it
        gt = jnp.where(v > tvl, 1, 0)
        eq = jnp.where(v == tvl, 1, 0)
        le = jnp.where(pv <= til, 1, 0)
        seli = gt + eq * le
        plsc.store_scatter(sel_v, [it * 128 + lanecnt], pv, mask=(seli == 1))
        lanecnt = lanecnt + seli
      return lanecnt

    lanecnt = lax.fori_loop(
        0, _VROWS // 2, sbody, jnp.zeros((16,), jnp.int32))

    off = base - halo

    def gbody(s, carry):
      psum, lcv = carry
      it = lax.iota(jnp.int32, 16)
      pv = plsc.load_gather(sel_v, [it * 128 + s])
      valid = lcv > s
      posle = c0_v[...]
      for r in range(_NT):
        cr = tcells_v[pl.ds(r * 16, 16)]
        posle = posle + jnp.where(pv >= cr, 1, 0)
      lidx = jnp.where(valid, pv - posle - off, 0)
      vals = plsc.load_gather(bce_v, [lidx])
      return psum + jnp.where(valid, vals, 0.0), lcv

    psum, _ = lax.fori_loop(
        0, 128, gbody, (jnp.zeros((16,), jnp.float32), lanecnt))

    part_v[...] = psum
    pltpu.sync_copy(part_v, out_hbm.at[wid])

  return _sc_hard


@jax.jit
def _run(pred, targets):
    t0 = targets[:, :, 0]
    t1 = targets[:, :, 1]
    gx = jnp.minimum(t0 // _STRIDE, _W - 1)
    gy = jnp.minimum(t1 // _STRIDE, _H - 1)
    offx = (t0.astype(jnp.float32)
            - (gx * _STRIDE).astype(jnp.float32)) / _STRIDE
    offy = (t1.astype(jnp.float32)
            - (gy * _STRIDE).astype(jnp.float32)) / _STRIDE
    cells = (jnp.arange(_BS, dtype=jnp.int32)[:, None] * (_H * _W)
             + gy * _W + gx).reshape(1, _NCELL)
    pr = pred.reshape(_BS * _H * _W, 3)
    pox = pr[:, 0].reshape(_BS * _H, _W)
    poy = pr[:, 1].reshape(_BS * _H, _W)
    pcls = pr[:, 2].reshape(_BS * _H, _W)

    smem = pl.BlockSpec(memory_space=pltpu.SMEM)
    vmem = pl.BlockSpec(memory_space=pltpu.VMEM)
    skeys, bcemap, ucells, params, sums, c0s = pl.pallas_call(
        _dense_kernel,
        out_shape=[
            jax.ShapeDtypeStruct((_BS * _H, _W), jnp.int32),
            jax.ShapeDtypeStruct((_BS * _H, _W), jnp.float32),
            jax.ShapeDtypeStruct((1, _NCELL), jnp.int32),
            jax.ShapeDtypeStruct((16,), jnp.int32),
            jax.ShapeDtypeStruct((8,), jnp.float32),
            jax.ShapeDtypeStruct((_NW,), jnp.int32),
        ],
        in_specs=[smem, smem, smem, smem, vmem, vmem, vmem, vmem],
        out_specs=[vmem, vmem, vmem, smem, smem, smem],
    )(gx, gy, offx, offy, cells, pcls, pox, poy)

    u = ucells.reshape(_BS, _NT)
    rel = u - jnp.arange(_BS, dtype=jnp.int32)[:, None] * (_H * _W)
    tc_lo = jnp.where((rel >= 0) & (rel < _CHUNK), u, _HUGE_I)
    tc_hi = jnp.where((rel >= _CHUNK) & (rel < 2 * _CHUNK), u, _HUGE_I)
    tcells = jnp.stack([tc_lo, tc_hi], axis=1)
    tcells_splat = jnp.repeat(tcells.reshape(_NW * _NT), 16)
    c0_splat = jnp.repeat(c0s, 16)
    params_splat = jnp.repeat(params[:2], 16)
    hard_parts = _make_sc_hard()(
        skeys.reshape(_NFLAT), bcemap.reshape(_NFLAT),
        tcells_splat, c0_splat, params_splat)
    hard_sum = jnp.sum(hard_parts)

    total_focal = sums[0]
    pos_focal = sums[1]
    npos_f = sums[2]
    reg_sum = sums[3]
    nneg_f = jnp.float32(_NFLAT) - npos_f
    den_pos = jnp.maximum(npos_f, 1.0)
    pos_loss = pos_focal / den_pos
    neg_loss = (total_focal - pos_focal) / jnp.maximum(nneg_f, 1.0)
    hard_neg_loss = hard_sum / den_pos
    cls_loss = pos_loss + neg_loss + hard_neg_loss
    reg = reg_sum / (den_pos * 2.0)
    reg_loss = jnp.where(npos_f == 0.0, 0.0, reg / den_pos)
    return _CLS_WEIGHT * cls_loss + _REG_WEIGHT * reg_loss


def kernel(pred, targets):
    return _run(pred, targets)
